# Initial kernel scaffold; baseline (speedup 1.0000x reference)
import jax
import jax.numpy as jnp
from jax.experimental import pallas as pl


def kernel(x, peak_positions, table):
    raise NotImplementedError("write your pallas kernel here")


# speedup vs baseline: 5.5729x; 5.5729x over previous
import functools
import math

import jax
import jax.numpy as jnp
from jax.experimental import pallas as pl

EMBED_DIM = 256
MAX_LEN = 2048
SEQ_LEN = 2048
BATCH = 64
PEAK_PAD = 64
SBLK = 128


def _pe_table(max_len, dim):
    position = jnp.arange(0, max_len, dtype=jnp.float32)[:, None]
    div_term = jnp.exp(
        jnp.arange(0, dim, 2, dtype=jnp.float32) * (-math.log(1000.0) / dim))
    pe = jnp.zeros((max_len, dim), dtype=jnp.float32)
    pe = pe.at[:, 0::2].set(jnp.sin(position * div_term))
    pe = pe.at[:, 1::2].set(jnp.cos(position * div_term))
    return pe


def _tc_body(peaks_ref, x_ref, pe_ref, tab_ref, out_ref):
    i = pl.program_id(0)
    s_ids = jax.lax.broadcasted_iota(jnp.int32, (SBLK, 1, 1), 0) + i * SBLK
    peaks = peaks_ref[...]
    hit = peaks[None, :, :] == s_ids
    mask = jnp.any(hit, axis=2)
    out_ref[...] = (
        x_ref[...]
        + pe_ref[...][:, None, :]
        + mask[:, :, None].astype(jnp.float32) * tab_ref[...][:, None, :]
    )


@functools.partial(jax.jit, static_argnames=("interpret",))
def _run(x, peaks, table, pe, interpret=False):
    seq, batch, dim = x.shape
    grid = (seq // SBLK,)
    return pl.pallas_call(
        _tc_body,
        grid=grid,
        in_specs=[
            pl.BlockSpec((BATCH, PEAK_PAD), lambda i: (0, 0)),
            pl.BlockSpec((SBLK, BATCH, EMBED_DIM), lambda i: (i, 0, 0)),
            pl.BlockSpec((SBLK, EMBED_DIM), lambda i: (i, 0)),
            pl.BlockSpec((SBLK, EMBED_DIM), lambda i: (i, 0)),
        ],
        out_specs=pl.BlockSpec((SBLK, BATCH, EMBED_DIM), lambda i: (i, 0, 0)),
        out_shape=jax.ShapeDtypeStruct((seq, batch, dim), jnp.float32),
        interpret=interpret,
    )(peaks, x, pe, table)


def kernel(x, peak_positions, table):
    seq, batch, dim = x.shape
    pe = _pe_table(seq, dim)
    peaks = jnp.pad(
        peak_positions.astype(jnp.int32),
        ((0, 0), (0, PEAK_PAD - peak_positions.shape[1])),
        constant_values=-1,
    )
    return _run(x, peaks, table, pe)
